# baseline (speedup 1.0000x reference)
import jax
import jax.numpy as jnp
from jax import lax
from jax.experimental import pallas as pl
from jax.experimental.pallas import tpu as pltpu
from jax.experimental.pallas import tpu_sc as plsc

_B = 16384
_S = 200
_N = _B * _S
_V = 1000000
_NC = 2
_NS = 16
_NW = _NC * _NS
_SEG = 2048
_SEG_PER_ROW = _B // _SEG
_NSEG = _S * _SEG_PER_ROW
_SEG_W = _NSEG // _NW
_PAIRS = _SEG_W // 2
_LANES = 16
_STAGE_CHUNK = 8192
_N_FULL_CHUNKS = _V // _STAGE_CHUNK
_STAGE_TAIL = _V - _N_FULL_CHUNKS * _STAGE_CHUNK
_N_STAGE_CHUNKS = _N_FULL_CHUNKS + 1
_STAGE_TRIPS = -(-_N_STAGE_CHUNKS // _NS)


def _gather_exp_body(x_hbm, table_hbm, out_hbm, stage_v, idx_a, idx_b,
                     rows_a, rows_b, etab_s, sem_g, sem_g2, sem_ia, sem_ib,
                     sem_oa, sem_ob):
    c = lax.axis_index("c")
    s = lax.axis_index("s")
    wid = s * _NC + c

    def _stage(off, size):
        pltpu.sync_copy(table_hbm.at[0, pl.ds(off, size)],
                        stage_v.at[pl.ds(0, size)])
        pltpu.sync_copy(stage_v.at[pl.ds(0, size)],
                        etab_s.at[pl.ds(off, size)])

    def stage_body(k, carry):
        ci = k * _NS + s

        @pl.when(ci < _N_FULL_CHUNKS)
        def _():
            _stage(ci * _STAGE_CHUNK, _STAGE_CHUNK)

        @pl.when(ci == _N_FULL_CHUNKS)
        def _():
            _stage(_N_FULL_CHUNKS * _STAGE_CHUNK, _STAGE_TAIL)

        return carry

    lax.fori_loop(0, _STAGE_TRIPS, stage_body, 0)
    plsc.subcore_barrier()

    base = wid * _SEG_W

    def _x_slice(q):
        r = q // _SEG_PER_ROW
        b0 = (q % _SEG_PER_ROW) * _SEG
        return x_hbm.at[pl.ds(r, 1), pl.ds(b0, _SEG)]

    def _out_slice(q):
        r = q // _SEG_PER_ROW
        b0 = (q % _SEG_PER_ROW) * _SEG
        return out_hbm.at[pl.ds(r, 1), pl.ds(b0, _SEG)]

    def _exp_seg(rows_c):
        def exp_body(vi, cc):
            sl = pl.ds(pl.multiple_of(vi * _LANES, _LANES), _LANES)
            rows_c[0, sl] = jnp.exp(rows_c[0, sl])
            return cc

        lax.fori_loop(0, _SEG // _LANES, exp_body, 0)

    def _issue_gather(q, qprev, idx_c, rows_c, sem_gc, sem_ic, sem_oc, j):
        pltpu.make_async_copy(_x_slice(q), idx_c, sem_ic).wait()

        @pl.when(j > 0)
        def _():
            pltpu.make_async_copy(rows_c, _out_slice(qprev), sem_oc).wait()

        pltpu.async_copy(etab_s.at[idx_c.at[0]], rows_c.at[0], sem_gc)

    def _finish_seg(q, qnext, idx_c, rows_c, sem_gc, sem_ic, sem_oc):
        pltpu.make_async_copy(etab_s.at[idx_c.at[0]], rows_c.at[0],
                              sem_gc).wait()

        @pl.when(qnext < base + _SEG_W)
        def _():
            pltpu.async_copy(_x_slice(qnext), idx_c, sem_ic)

        _exp_seg(rows_c)
        pltpu.async_copy(rows_c, _out_slice(q), sem_oc)

    pltpu.async_copy(_x_slice(base), idx_a, sem_ia)
    pltpu.async_copy(_x_slice(base + 1), idx_b, sem_ib)

    def pair_body(j, carry):
        q0 = base + 2 * j
        _issue_gather(q0, q0 - 2, idx_a, rows_a, sem_g, sem_ia, sem_oa, j)
        _issue_gather(q0 + 1, q0 - 1, idx_b, rows_b, sem_g2, sem_ib,
                      sem_ob, j)
        _finish_seg(q0, q0 + 2, idx_a, rows_a, sem_g, sem_ia, sem_oa)
        _finish_seg(q0 + 1, q0 + 3, idx_b, rows_b, sem_g2, sem_ib, sem_ob)
        return carry

    lax.fori_loop(0, _PAIRS, pair_body, 0)
    pltpu.make_async_copy(rows_a, _out_slice(base + _SEG_W - 2),
                          sem_oa).wait()
    pltpu.make_async_copy(rows_b, _out_slice(base + _SEG_W - 1),
                          sem_ob).wait()


def kernel(x, table):
    xt = x.T
    tt = table.T
    mesh = plsc.VectorSubcoreMesh(core_axis_name="c", subcore_axis_name="s")
    fn = pl.kernel(
        _gather_exp_body,
        out_type=jax.ShapeDtypeStruct((_S, _B), jnp.float32),
        mesh=mesh,
        scratch_types=[
            pltpu.VMEM((_STAGE_CHUNK,), jnp.float32),
            pltpu.VMEM((1, _SEG), jnp.int32),
            pltpu.VMEM((1, _SEG), jnp.int32),
            pltpu.VMEM((1, _SEG), jnp.float32),
            pltpu.VMEM((1, _SEG), jnp.float32),
            pltpu.VMEM_SHARED((_V,), jnp.float32),
            pltpu.SemaphoreType.DMA,
            pltpu.SemaphoreType.DMA,
            pltpu.SemaphoreType.DMA,
            pltpu.SemaphoreType.DMA,
            pltpu.SemaphoreType.DMA,
            pltpu.SemaphoreType.DMA,
        ],
    )
    return fn(xt, tt).T


# speedup vs baseline: 1.0476x; 1.0476x over previous
import jax
import jax.numpy as jnp
from jax import lax
from jax.experimental import pallas as pl
from jax.experimental.pallas import tpu as pltpu
from jax.experimental.pallas import tpu_sc as plsc

_B = 16384
_S = 200
_N = _B * _S
_V = 1000000
_NC = 2
_NS = 16
_NW = _NC * _NS
_SEG = 2048
_SEG_PER_ROW = _B // _SEG
_NSEG = _S * _SEG_PER_ROW
_SEG_W = _NSEG // _NW
_PAIRS = _SEG_W // 2
_LANES = 16
_STAGE_CHUNK = 8192
_N_FULL_CHUNKS = _V // _STAGE_CHUNK
_STAGE_TAIL = _V - _N_FULL_CHUNKS * _STAGE_CHUNK
_N_STAGE_CHUNKS = _N_FULL_CHUNKS + 1
_STAGE_TRIPS = -(-_N_STAGE_CHUNKS // _NS)


def _gather_exp_body(x_hbm, table_hbm, out_hbm, stage_v, idx_a, idx_b,
                     rows_a, rows_b, etab_s, sem_g, sem_g2, sem_ia, sem_ib,
                     sem_oa, sem_ob):
    c = lax.axis_index("c")
    s = lax.axis_index("s")
    wid = s * _NC + c

    def _stage(off, size):
        pltpu.sync_copy(table_hbm.at[0, pl.ds(off, size)],
                        stage_v.at[pl.ds(0, size)])

        def exp_body(vi, cc):
            sl = pl.ds(pl.multiple_of(vi * _LANES, _LANES), _LANES)
            stage_v[sl] = jnp.exp(stage_v[sl])
            return cc

        lax.fori_loop(0, size // _LANES, exp_body, 0)
        pltpu.sync_copy(stage_v.at[pl.ds(0, size)],
                        etab_s.at[pl.ds(off, size)])

    def stage_body(k, carry):
        ci = k * _NS + s

        @pl.when(ci < _N_FULL_CHUNKS)
        def _():
            _stage(ci * _STAGE_CHUNK, _STAGE_CHUNK)

        @pl.when(ci == _N_FULL_CHUNKS)
        def _():
            _stage(_N_FULL_CHUNKS * _STAGE_CHUNK, _STAGE_TAIL)

        return carry

    lax.fori_loop(0, _STAGE_TRIPS, stage_body, 0)
    plsc.subcore_barrier()

    base = wid * _SEG_W

    def _x_slice(q):
        r = q // _SEG_PER_ROW
        b0 = (q % _SEG_PER_ROW) * _SEG
        return x_hbm.at[pl.ds(r, 1), pl.ds(b0, _SEG)]

    def _out_slice(q):
        r = q // _SEG_PER_ROW
        b0 = (q % _SEG_PER_ROW) * _SEG
        return out_hbm.at[pl.ds(r, 1), pl.ds(b0, _SEG)]

    def _issue_gather(q, qprev, idx_c, rows_c, sem_gc, sem_ic, sem_oc, j):
        pltpu.make_async_copy(_x_slice(q), idx_c, sem_ic).wait()

        @pl.when(j > 0)
        def _():
            pltpu.make_async_copy(rows_c, _out_slice(qprev), sem_oc).wait()

        pltpu.async_copy(etab_s.at[idx_c.at[0]], rows_c.at[0], sem_gc)

    def _finish_seg(q, qnext, idx_c, rows_c, sem_gc, sem_ic, sem_oc):
        pltpu.make_async_copy(etab_s.at[idx_c.at[0]], rows_c.at[0],
                              sem_gc).wait()

        @pl.when(qnext < base + _SEG_W)
        def _():
            pltpu.async_copy(_x_slice(qnext), idx_c, sem_ic)

        pltpu.async_copy(rows_c, _out_slice(q), sem_oc)

    pltpu.async_copy(_x_slice(base), idx_a, sem_ia)
    pltpu.async_copy(_x_slice(base + 1), idx_b, sem_ib)

    def pair_body(j, carry):
        q0 = base + 2 * j
        _issue_gather(q0, q0 - 2, idx_a, rows_a, sem_g, sem_ia, sem_oa, j)
        _issue_gather(q0 + 1, q0 - 1, idx_b, rows_b, sem_g2, sem_ib,
                      sem_ob, j)
        _finish_seg(q0, q0 + 2, idx_a, rows_a, sem_g, sem_ia, sem_oa)
        _finish_seg(q0 + 1, q0 + 3, idx_b, rows_b, sem_g2, sem_ib, sem_ob)
        return carry

    lax.fori_loop(0, _PAIRS, pair_body, 0)
    pltpu.make_async_copy(rows_a, _out_slice(base + _SEG_W - 2),
                          sem_oa).wait()
    pltpu.make_async_copy(rows_b, _out_slice(base + _SEG_W - 1),
                          sem_ob).wait()


def kernel(x, table):
    xt = x.T
    tt = table.T
    mesh = plsc.VectorSubcoreMesh(core_axis_name="c", subcore_axis_name="s")
    fn = pl.kernel(
        _gather_exp_body,
        out_type=jax.ShapeDtypeStruct((_S, _B), jnp.float32),
        mesh=mesh,
        scratch_types=[
            pltpu.VMEM((_STAGE_CHUNK,), jnp.float32),
            pltpu.VMEM((1, _SEG), jnp.int32),
            pltpu.VMEM((1, _SEG), jnp.int32),
            pltpu.VMEM((1, _SEG), jnp.float32),
            pltpu.VMEM((1, _SEG), jnp.float32),
            pltpu.VMEM_SHARED((_V,), jnp.float32),
            pltpu.SemaphoreType.DMA,
            pltpu.SemaphoreType.DMA,
            pltpu.SemaphoreType.DMA,
            pltpu.SemaphoreType.DMA,
            pltpu.SemaphoreType.DMA,
            pltpu.SemaphoreType.DMA,
        ],
    )
    return fn(xt, tt).T


# speedup vs baseline: 1.1547x; 1.1022x over previous
import jax
import jax.numpy as jnp
from jax import lax
from jax.experimental import pallas as pl
from jax.experimental.pallas import tpu as pltpu
from jax.experimental.pallas import tpu_sc as plsc

_B = 16384
_S = 200
_N = _B * _S
_V = 1000000
_NC = 2
_NS = 16
_NW = _NC * _NS
_SEG = 2048
_SEG_PER_ROW = _B // _SEG
_NSEG = _S * _SEG_PER_ROW
_SEG_W = _NSEG // _NW
_PAIRS = _SEG_W // 2
_LANES = 16
_STAGE_CHUNK = 8192
_N_FULL_CHUNKS = _V // _STAGE_CHUNK
_STAGE_TAIL = _V - _N_FULL_CHUNKS * _STAGE_CHUNK
_N_STAGE_CHUNKS = _N_FULL_CHUNKS + 1
_STAGE_TRIPS = -(-_N_STAGE_CHUNKS // _NS)


def _gather_exp_body(x_hbm, table_hbm, out_hbm, stage_v, idx_a, idx_b,
                     rows_a, rows_b, etab_s, sem_g, sem_g2, sem_ia, sem_ib,
                     sem_oa, sem_ob):
    c = lax.axis_index("c")
    s = lax.axis_index("s")
    wid = s * _NC + c

    base = wid * _SEG_W

    def _x_slice(q):
        r = q // _SEG_PER_ROW
        b0 = (q % _SEG_PER_ROW) * _SEG
        return x_hbm.at[pl.ds(r, 1), pl.ds(b0, _SEG)]

    pltpu.async_copy(_x_slice(base), idx_a, sem_ia)

    def _stage(off, size):
        pltpu.sync_copy(table_hbm.at[0, pl.ds(off, size)],
                        stage_v.at[pl.ds(0, size)])

        def exp_body(vi, cc):
            sl = pl.ds(pl.multiple_of(vi * _LANES, _LANES), _LANES)
            stage_v[sl] = jnp.exp(stage_v[sl])
            return cc

        lax.fori_loop(0, size // _LANES, exp_body, 0)
        pltpu.sync_copy(stage_v.at[pl.ds(0, size)],
                        etab_s.at[pl.ds(off, size)])

    def stage_body(k, carry):
        ci = k * _NS + s

        @pl.when(ci < _N_FULL_CHUNKS)
        def _():
            _stage(ci * _STAGE_CHUNK, _STAGE_CHUNK)

        @pl.when(ci == _N_FULL_CHUNKS)
        def _():
            _stage(_N_FULL_CHUNKS * _STAGE_CHUNK, _STAGE_TAIL)

        return carry

    lax.fori_loop(0, _STAGE_TRIPS, stage_body, 0)
    plsc.subcore_barrier()

    def _out_slice(q):
        r = q // _SEG_PER_ROW
        b0 = (q % _SEG_PER_ROW) * _SEG
        return out_hbm.at[pl.ds(r, 1), pl.ds(b0, _SEG)]

    def _run_seg(q, qnext, qprev, idx_c, rows_c, idx_n, sem_ic, sem_in,
                 sem_oc, j):
        pltpu.make_async_copy(_x_slice(q), idx_c, sem_ic).wait()

        @pl.when(qnext < base + _SEG_W)
        def _():
            pltpu.async_copy(_x_slice(qnext), idx_n, sem_in)

        @pl.when(j > 0)
        def _():
            pltpu.make_async_copy(rows_c, _out_slice(qprev), sem_oc).wait()

        pltpu.async_copy(etab_s.at[idx_c.at[0]], rows_c.at[0], sem_g).wait()
        pltpu.async_copy(rows_c, _out_slice(q), sem_oc)

    def pair_body(j, carry):
        q0 = base + 2 * j
        _run_seg(q0, q0 + 1, q0 - 2, idx_a, rows_a, idx_b, sem_ia, sem_ib,
                 sem_oa, j)
        _run_seg(q0 + 1, q0 + 2, q0 - 1, idx_b, rows_b, idx_a, sem_ib,
                 sem_ia, sem_ob, j)
        return carry

    lax.fori_loop(0, _PAIRS, pair_body, 0)
    pltpu.make_async_copy(rows_a, _out_slice(base + _SEG_W - 2),
                          sem_oa).wait()
    pltpu.make_async_copy(rows_b, _out_slice(base + _SEG_W - 1),
                          sem_ob).wait()


def kernel(x, table):
    xt = x.T
    tt = table.T
    mesh = plsc.VectorSubcoreMesh(core_axis_name="c", subcore_axis_name="s")
    fn = pl.kernel(
        _gather_exp_body,
        out_type=jax.ShapeDtypeStruct((_S, _B), jnp.float32),
        mesh=mesh,
        scratch_types=[
            pltpu.VMEM((_STAGE_CHUNK,), jnp.float32),
            pltpu.VMEM((1, _SEG), jnp.int32),
            pltpu.VMEM((1, _SEG), jnp.int32),
            pltpu.VMEM((1, _SEG), jnp.float32),
            pltpu.VMEM((1, _SEG), jnp.float32),
            pltpu.VMEM_SHARED((_V,), jnp.float32),
            pltpu.SemaphoreType.DMA,
            pltpu.SemaphoreType.DMA,
            pltpu.SemaphoreType.DMA,
            pltpu.SemaphoreType.DMA,
            pltpu.SemaphoreType.DMA,
            pltpu.SemaphoreType.DMA,
        ],
    )
    return fn(xt, tt).T
